# Initial kernel scaffold; baseline (speedup 1.0000x reference)
import jax
import jax.numpy as jnp
from jax.experimental import pallas as pl


def kernel(coords, assigned_params, num_atoms):
    raise NotImplementedError("write your pallas kernel here")


# speedup vs baseline: 31.4837x; 31.4837x over previous
import jax
import jax.numpy as jnp
from jax.experimental import pallas as pl

B = 4
A = 2048
BOX = 80
RES = 1.0
EPS_IN = 6.5
EPS_OUT = 79.0
ION = 1.0
WAT = 1.4
ASIG = 2.0
KAPPA02 = 0.8486
QCONV = 7046.52
HALF = 3
N_ITER = 20

XB = 16
NX = BOX // XB
ADDS = (0.0, ION, WAT, ION + WAT)


def _splat_kernel(xs_ref, ys_ref, sigr_ref, zc_ref, sigc_ref, chc_ref,
                  mkc_ref, eps_ref, q_ref):
    xi = pl.program_id(1)
    xs = xs_ref[0]
    ys = ys_ref[0]
    sigr = sigr_ref[0]
    zc = zc_ref[0]
    sigc = sigc_ref[0]
    chc = chc_ref[0]
    mkc = mkc_ref[0]

    x_base = (xi * XB).astype(jnp.float32)
    ix = jax.lax.broadcasted_iota(jnp.int32, (XB, A), 0).astype(jnp.float32) + x_base
    iy = jax.lax.broadcasted_iota(jnp.int32, (BOX, A), 0).astype(jnp.float32)
    iz = jax.lax.broadcasted_iota(jnp.int32, (A, BOX), 1).astype(jnp.float32)

    x0 = jnp.floor(xs)
    y0 = jnp.floor(ys)
    z0 = jnp.floor(zc)
    dx2 = (ix - xs) ** 2
    dy2 = (iy - ys) ** 2
    dz2 = (iz - zc) ** 2
    wx = ((ix >= x0 - 3.0) & (ix <= x0 + 3.0)).astype(jnp.float32)
    wy = ((iy >= y0 - 3.0) & (iy <= y0 + 3.0)).astype(jnp.float32)
    wz = ((iz >= z0 - 3.0) & (iz <= z0 + 3.0)).astype(jnp.float32) * mkc

    scale = EPS_IN - EPS_OUT
    for c in range(4):
        invr = 1.0 / (ASIG * (sigr + ADDS[c]))
        invc = 1.0 / (ASIG * (sigc + ADDS[c]))
        gx = jnp.exp(-dx2 * (invr * invr)) * wx
        gy = jnp.exp(-dy2 * (invr * invr)) * wy
        gz = jnp.exp(-dz2 * (invc * invc)) * wz
        kt = (gx[:, None, :] * gy[None, :, :]).reshape(XB * BOX, A)
        rho = jnp.dot(kt, gz, preferred_element_type=jnp.float32)
        rho = rho.reshape(XB, BOX, BOX)
        eps_ref[0, c] = jnp.clip(rho, 0.0, 1.0) * scale + EPS_OUT

    qx = jnp.maximum(1.0 - jnp.abs(ix - xs), 0.0)
    qy = jnp.maximum(1.0 - jnp.abs(iy - ys), 0.0)
    qz = jnp.maximum(1.0 - jnp.abs(iz - zc), 0.0) * (chc * mkc)
    ktq = (qx[:, None, :] * qy[None, :, :]).reshape(XB * BOX, A)
    accq = jnp.dot(ktq, qz, preferred_element_type=jnp.float32)
    q_ref[0] = accq.reshape(XB, BOX, BOX) * QCONV


def _jacobi_kernel(q_ref, eps_ref, phi_ref):
    den = (eps_ref[0, 0] + jnp.roll(eps_ref[0, 0], 1, 0)
           + eps_ref[0, 1] + jnp.roll(eps_ref[0, 1], 1, 1)
           + eps_ref[0, 2] + jnp.roll(eps_ref[0, 2], 1, 2)
           + KAPPA02 * RES * RES)
    inv_den = 1.0 / den

    def body(_, phi):
        num = (eps_ref[0, 0] * jnp.roll(phi, -1, 0)
               + jnp.roll(eps_ref[0, 0] * phi, 1, 0)
               + eps_ref[0, 1] * jnp.roll(phi, -1, 1)
               + jnp.roll(eps_ref[0, 1] * phi, 1, 1)
               + eps_ref[0, 2] * jnp.roll(phi, -1, 2)
               + jnp.roll(eps_ref[0, 2] * phi, 1, 2)
               + q_ref[0] * RES)
        return num * inv_den

    phi0 = jnp.zeros((BOX, BOX, BOX), jnp.float32)
    phi_ref[0] = jax.lax.fori_loop(0, N_ITER, body, phi0)


def kernel(coords, assigned_params, num_atoms):
    xyz = coords.reshape(B, A, 3) / RES
    xs, ys, zs = xyz[..., 0], xyz[..., 1], xyz[..., 2]
    charges = assigned_params[..., 0]
    sigmas = assigned_params[..., 1]
    mask = (jnp.arange(A)[None, :] < num_atoms[:, None]).astype(jnp.float32)

    eps, q = pl.pallas_call(
        _splat_kernel,
        grid=(B, NX),
        in_specs=[pl.BlockSpec((1, 1, A), lambda b, x: (b, 0, 0))] * 3
        + [pl.BlockSpec((1, A, 1), lambda b, x: (b, 0, 0))] * 4,
        out_specs=[
            pl.BlockSpec((1, 4, XB, BOX, BOX), lambda b, x: (b, 0, x, 0, 0)),
            pl.BlockSpec((1, XB, BOX, BOX), lambda b, x: (b, x, 0, 0)),
        ],
        out_shape=[
            jax.ShapeDtypeStruct((B, 4, BOX, BOX, BOX), jnp.float32),
            jax.ShapeDtypeStruct((B, BOX, BOX, BOX), jnp.float32),
        ],
    )(xs[:, None], ys[:, None], sigmas[:, None], zs[..., None],
      sigmas[..., None], charges[..., None], mask[..., None])

    phi = pl.pallas_call(
        _jacobi_kernel,
        grid=(B,),
        in_specs=[
            pl.BlockSpec((1, BOX, BOX, BOX), lambda b: (b, 0, 0, 0)),
            pl.BlockSpec((1, 3, BOX, BOX, BOX), lambda b: (b, 0, 0, 0, 0)),
        ],
        out_specs=pl.BlockSpec((1, BOX, BOX, BOX), lambda b: (b, 0, 0, 0)),
        out_shape=jax.ShapeDtypeStruct((B, BOX, BOX, BOX), jnp.float32),
    )(q, eps[:, :3])

    return (q, eps, phi)
